# baseline (speedup 1.0000x reference)
import functools

import jax
import jax.numpy as jnp
from jax import lax
from jax.experimental import pallas as pl
from jax.experimental.pallas import tpu as pltpu
from jax.experimental.pallas import tpu_sc as plsc

VOCAB = 1000000
SYN_NUM = 10
DPAD = 16
B = 4096
L = 200
N = B * L

NUM_CORES = 2
NUM_SUBCORES = 16
NUM_WORKERS = NUM_CORES * NUM_SUBCORES
B_PER_W = N // NUM_WORKERS
CHUNK = 2560
NUM_CHUNKS = B_PER_W // CHUNK


def _body(idx_hbm, syn_hbm, mask_hbm, syns_out, mask_out,
          idx_v, syn_v, mask_v, sem_s, sem_m):
    wid = lax.axis_index("s") * NUM_CORES + lax.axis_index("c")
    base_w = wid * B_PER_W
    pltpu.sync_copy(idx_hbm.at[pl.ds(base_w, B_PER_W)], idx_v)
    for j in range(NUM_CHUNKS):
        base = base_w + j * CHUNK
        idx_sl = idx_v.at[pl.ds(j * CHUNK, CHUNK)]
        cp_s = pltpu.async_copy(syn_hbm.at[idx_sl], syn_v, sem_s)
        cp_m = pltpu.async_copy(mask_hbm.at[idx_sl], mask_v, sem_m)
        cp_s.wait()
        pltpu.sync_copy(syn_v, syns_out.at[pl.ds(base, CHUNK)])
        cp_m.wait()
        pltpu.sync_copy(mask_v, mask_out.at[pl.ds(base, CHUNK)])


_lookup = functools.partial(
    pl.kernel,
    out_type=(
        jax.ShapeDtypeStruct((N, DPAD), jnp.int32),
        jax.ShapeDtypeStruct((N, DPAD), jnp.float32),
    ),
    mesh=plsc.VectorSubcoreMesh(core_axis_name="c", subcore_axis_name="s"),
    scratch_types=[
        pltpu.VMEM((B_PER_W,), jnp.int32),
        pltpu.VMEM((CHUNK, DPAD), jnp.int32),
        pltpu.VMEM((CHUNK, DPAD), jnp.float32),
        pltpu.SemaphoreType.DMA,
        pltpu.SemaphoreType.DMA,
    ],
    compiler_params=pltpu.CompilerParams(use_tc_tiling_on_sc=False),
)(_body)


@jax.jit
def kernel(idx, syn_table, mask_table):
    idx_flat = idx.reshape(N)
    pad = ((0, 0), (0, DPAD - SYN_NUM))
    syn16 = jnp.pad(syn_table.astype(jnp.int32), pad)
    mask16 = jnp.pad(mask_table, pad)
    syns16, m16 = _lookup(idx_flat, syn16, mask16)
    syns = syns16[:, :SYN_NUM].reshape(B, L, SYN_NUM)
    mask = m16[:, :SYN_NUM].reshape(B, L, SYN_NUM)
    return syns, mask


# speedup vs baseline: 1.2087x; 1.2087x over previous
import functools

import jax
import jax.numpy as jnp
from jax import lax
from jax.experimental import pallas as pl
from jax.experimental.pallas import tpu as pltpu
from jax.experimental.pallas import tpu_sc as plsc

VOCAB = 1000000
SYN_NUM = 10
DPAD = 16
B = 4096
L = 200
N = B * L

NUM_CORES = 2
NUM_SUBCORES = 16
NUM_WORKERS = NUM_CORES * NUM_SUBCORES
B_PER_W = N // NUM_WORKERS
CHUNK = 1600
NUM_CHUNKS = B_PER_W // CHUNK
LANES = 16
VECS = CHUNK // LANES


def _body(idx_hbm, syn_hbm, mask_hbm, syns_out, mask_out,
          idx_v, syn_v, mask_v, syn_t, mask_t, sem_s, sem_m):
    syn2d = syn_hbm
    mask2d = mask_hbm
    wid = lax.axis_index("s") * NUM_CORES + lax.axis_index("c")
    base_w = wid * B_PER_W
    pltpu.sync_copy(idx_hbm.at[pl.ds(base_w, B_PER_W)], idx_v)
    lane = lax.iota(jnp.int32, LANES)

    def chunk_step(j, carry):
        base = base_w + j * CHUNK
        idx_sl = idx_v.at[pl.ds(j * CHUNK, CHUNK)]
        cp_s = pltpu.async_copy(syn2d.at[idx_sl], syn_v, sem_s)
        cp_m = pltpu.async_copy(mask2d.at[idx_sl], mask_v, sem_m)
        cp_s.wait()
        cp_m.wait()

        def vec_step(i, carry2):
            rows = lane + i * LANES
            for c in range(SYN_NUM):
                cols = jnp.full((LANES,), c, jnp.int32)
                sv = plsc.load_gather(syn_v, [rows, cols])
                syn_t[c, pl.ds(i * LANES, LANES)] = sv
                mv = plsc.load_gather(mask_v, [rows, cols])
                mask_t[c, pl.ds(i * LANES, LANES)] = mv
            return carry2

        lax.fori_loop(0, VECS, vec_step, 0)
        for c in range(SYN_NUM):
            pltpu.sync_copy(syn_t.at[c], syns_out.at[c, pl.ds(base, CHUNK)])
            pltpu.sync_copy(mask_t.at[c], mask_out.at[c, pl.ds(base, CHUNK)])
        return carry

    lax.fori_loop(0, NUM_CHUNKS, chunk_step, 0)


_lookup = functools.partial(
    pl.kernel,
    out_type=(
        jax.ShapeDtypeStruct((SYN_NUM, N), jnp.int32),
        jax.ShapeDtypeStruct((SYN_NUM, N), jnp.float32),
    ),
    mesh=plsc.VectorSubcoreMesh(core_axis_name="c", subcore_axis_name="s"),
    scratch_types=[
        pltpu.VMEM((B_PER_W,), jnp.int32),
        pltpu.VMEM((CHUNK, DPAD), jnp.int32),
        pltpu.VMEM((CHUNK, DPAD), jnp.float32),
        pltpu.VMEM((SYN_NUM, CHUNK), jnp.int32),
        pltpu.VMEM((SYN_NUM, CHUNK), jnp.float32),
        pltpu.SemaphoreType.DMA,
        pltpu.SemaphoreType.DMA,
    ],
    compiler_params=pltpu.CompilerParams(
        use_tc_tiling_on_sc=False, needs_layout_passes=False),
)(_body)


@jax.jit
def kernel(idx, syn_table, mask_table):
    idx_t = idx.T.reshape(N)
    zi = jnp.zeros((DPAD - SYN_NUM, VOCAB), jnp.int32)
    zf = jnp.zeros((DPAD - SYN_NUM, VOCAB), jnp.float32)
    syn_c = jnp.concatenate([syn_table.T.astype(jnp.int32), zi], axis=0)
    mask_c = jnp.concatenate([mask_table.T, zf], axis=0)
    syn_c, mask_c = jax.lax.optimization_barrier((syn_c, mask_c))
    syn16 = syn_c.T.reshape(VOCAB * DPAD).reshape(VOCAB, DPAD)
    mask16 = mask_c.T.reshape(VOCAB * DPAD).reshape(VOCAB, DPAD)
    syn16, mask16 = jax.lax.optimization_barrier((syn16, mask16))
    syns_pl, mask_pl = _lookup(idx_t, syn16, mask16)
    syns = jnp.transpose(syns_pl.reshape(SYN_NUM, L, B), (2, 1, 0))
    mask = jnp.transpose(mask_pl.reshape(SYN_NUM, L, B), (2, 1, 0))
    return syns, mask


# speedup vs baseline: 1.5589x; 1.2897x over previous
import functools

import jax
import jax.numpy as jnp
from jax import lax
from jax.experimental import pallas as pl
from jax.experimental.pallas import tpu as pltpu
from jax.experimental.pallas import tpu_sc as plsc

VOCAB = 1000000
SYN_NUM = 10
B = 4096
L = 200
N = B * L

VIEW_W = 16
VIEW_ROWS = VOCAB * SYN_NUM // VIEW_W

NUM_CORES = 2
NUM_SUBCORES = 16
NUM_WORKERS = NUM_CORES * NUM_SUBCORES
B_PER_W = N // NUM_WORKERS
CHUNK = 1024
NUM_CHUNKS = B_PER_W // CHUNK
LANES = 16
VECS = CHUNK // LANES


def _body(idx_hbm, syn_hbm, mask_hbm, syns_out, mask_out,
          idx_v, klo_v, khi_v, syn_b, mask_b, syn_t, mask_t, sem_s, sem_m):
    wid = lax.axis_index("s") * NUM_CORES + lax.axis_index("c")
    base_w = wid * B_PER_W
    pltpu.sync_copy(idx_hbm.at[pl.ds(base_w, B_PER_W)], idx_v)
    lane = lax.iota(jnp.int32, LANES)

    def chunk_step(j, carry):
        base = base_w + j * CHUNK

        def idx_step(i, carry2):
            v = idx_v[pl.ds(j * CHUNK + i * LANES, LANES)]
            klo = (v * SYN_NUM) >> 4
            klo_v[pl.ds(i * LANES, LANES)] = klo
            khi_v[pl.ds(i * LANES, LANES)] = jnp.minimum(klo + 1, VIEW_ROWS - 1)
            return carry2

        lax.fori_loop(0, VECS, idx_step, 0)
        cp_sl = pltpu.async_copy(syn_hbm.at[klo_v], syn_b.at[0], sem_s)
        cp_sh = pltpu.async_copy(syn_hbm.at[khi_v], syn_b.at[1], sem_s)
        cp_ml = pltpu.async_copy(mask_hbm.at[klo_v], mask_b.at[0], sem_m)
        cp_mh = pltpu.async_copy(mask_hbm.at[khi_v], mask_b.at[1], sem_m)
        cp_sl.wait()
        cp_sh.wait()
        cp_ml.wait()
        cp_mh.wait()

        def vec_step(i, carry2):
            rows = lane + i * LANES
            v = idx_v[pl.ds(j * CHUNK + i * LANES, LANES)]
            off = (v * SYN_NUM) & 15
            for c in range(SYN_NUM):
                t = off + c
                sel = t >> 4
                col = t & 15
                sv = plsc.load_gather(syn_b, [sel, rows, col])
                syn_t[c, pl.ds(i * LANES, LANES)] = sv.astype(jnp.int32)
                mv = plsc.load_gather(mask_b, [sel, rows, col])
                mask_t[c, pl.ds(i * LANES, LANES)] = mv
            return carry2

        lax.fori_loop(0, VECS, vec_step, 0)
        for c in range(SYN_NUM):
            pltpu.sync_copy(syn_t.at[c], syns_out.at[c, pl.ds(base, CHUNK)])
            pltpu.sync_copy(mask_t.at[c], mask_out.at[c, pl.ds(base, CHUNK)])
        return carry

    lax.fori_loop(0, NUM_CHUNKS, chunk_step, 0)


_lookup = functools.partial(
    pl.kernel,
    out_type=(
        jax.ShapeDtypeStruct((SYN_NUM, N), jnp.int32),
        jax.ShapeDtypeStruct((SYN_NUM, N), jnp.float32),
    ),
    mesh=plsc.VectorSubcoreMesh(core_axis_name="c", subcore_axis_name="s"),
    scratch_types=[
        pltpu.VMEM((B_PER_W,), jnp.int32),
        pltpu.VMEM((CHUNK,), jnp.int32),
        pltpu.VMEM((CHUNK,), jnp.int32),
        pltpu.VMEM((2, CHUNK, VIEW_W), jnp.float32),
        pltpu.VMEM((2, CHUNK, VIEW_W), jnp.float32),
        pltpu.VMEM((SYN_NUM, CHUNK), jnp.int32),
        pltpu.VMEM((SYN_NUM, CHUNK), jnp.float32),
        pltpu.SemaphoreType.DMA,
        pltpu.SemaphoreType.DMA,
    ],
    compiler_params=pltpu.CompilerParams(
        use_tc_tiling_on_sc=False, needs_layout_passes=False),
)(_body)


@jax.jit
def kernel(idx, syn_table, mask_table):
    idx_t = idx.T.reshape(N)
    syn_v16 = syn_table.reshape(VIEW_ROWS, VIEW_W)
    mask_v16 = mask_table.reshape(VIEW_ROWS, VIEW_W)
    syns_pl, mask_pl = _lookup(idx_t, syn_v16, mask_v16)
    syns = jnp.transpose(syns_pl.reshape(SYN_NUM, L, B), (2, 1, 0))
    mask = jnp.transpose(mask_pl.reshape(SYN_NUM, L, B), (2, 1, 0))
    return syns, mask
